# baseline (speedup 1.0000x reference)
import functools

import jax
import jax.numpy as jnp
from jax import lax
from jax.experimental import pallas as pl
from jax.experimental.pallas import tpu as pltpu
from jax.experimental.pallas import tpu_sc as plsc

_NT = 50000
_B = 1024
_BIT = 32
_NC = 10
_TW = 128
_PK = 4
_NTF = _NT // _PK
_YW = _PK * _NC
_ETA = 0.001
_NJ = 25
_CBF = _NTF // _NJ
_B4 = _PK * _B

_LOG2E = 1.4426950408889634
_LN2 = 0.6931471805599453
_DN = (((1,), (1,)), ((), ()))

_NW = 32
_BPW = _B // _NW


def _sc_gather(U4, tabY, ind4):
    mesh = plsc.VectorSubcoreMesh(core_axis_name="c", subcore_axis_name="s")

    @functools.partial(
        pl.kernel,
        mesh=mesh,
        out_type=(
            jax.ShapeDtypeStruct((_B, _TW), jnp.float32),
            jax.ShapeDtypeStruct((_B, _TW), jnp.float32),
        ),
        scratch_types=[
            pltpu.VMEM((_BPW,), jnp.int32),
            pltpu.VMEM((_BPW, _TW), jnp.float32),
            pltpu.VMEM((_BPW, _TW), jnp.float32),
            pltpu.SemaphoreType.DMA,
        ],
    )
    def gather_k(u_hbm, ty_hbm, idx_hbm, gu_hbm, gy_hbm, idx_v, ru_v, ry_v, sem):
        wid = lax.axis_index("s") * 2 + lax.axis_index("c")
        base = wid * _BPW
        pltpu.sync_copy(idx_hbm.at[pl.ds(base, _BPW)], idx_v)
        pltpu.async_copy(u_hbm.at[idx_v], ru_v, sem).wait()
        pltpu.sync_copy(ru_v, gu_hbm.at[pl.ds(base, _BPW)])
        pltpu.async_copy(ty_hbm.at[idx_v], ry_v, sem).wait()
        pltpu.sync_copy(ry_v, gy_hbm.at[pl.ds(base, _BPW)])

    return gather_k(U4, tabY, ind4)


def _colsums(ip, sd):
    a = jnp.abs(ip)
    lg = jnp.log2(1.0 + jnp.exp2(a * (-_LOG2E)))
    cs_g = (jnp.sum(lg, axis=0, keepdims=True) * _LN2
            + 0.5 * jnp.sum(a, axis=0, keepdims=True))
    cs_sip = jnp.sum(jnp.where(sd > 0, ip, 0.0), axis=0, keepdims=True)
    return cs_g, cs_sip


def _main_kernel(uh4_ref, y4_ref, ush4_ref, U4_ref, Y4_ref, out_ref, tab_ref):
    j = pl.program_id(0)
    uh4 = uh4_ref[...]
    y4 = y4_ref[...]
    U4b = U4_ref[0]
    Y4b = Y4_ref[0]
    tab_ref[0, :, 0:_YW] = Y4b
    ip = jax.lax.dot_general(uh4, U4b.astype(jnp.bfloat16), _DN,
                             preferred_element_type=jnp.float32)
    sd = jax.lax.dot_general(y4, Y4b.astype(jnp.bfloat16), _DN,
                             preferred_element_type=jnp.float32)
    cs_g, cs_sip = _colsums(ip, sd)
    cs_ip = jax.lax.dot_general(ush4_ref[...], U4b, _DN,
                                preferred_element_type=jnp.float32)
    contrib = jnp.sum(cs_g + 0.5 * cs_ip - cs_sip)

    @pl.when(j == 0)
    def _first():
        out_ref[...] = jnp.full((1, 1), contrib, jnp.float32)

    @pl.when(j != 0)
    def _rest():
        out_ref[...] = out_ref[...] + contrib


def _corr_kernel(u_ref, y_ref, indc_ref, indr_ref, gu_ref, gy_ref,
                 acc_ref, out_ref):
    u = u_ref[...]
    uh = u * 0.5
    uh16 = uh.astype(jnp.bfloat16)
    y16 = y_ref[...].astype(jnp.bfloat16)
    ush = jnp.sum(uh, axis=0, keepdims=True)
    ind_c = indc_ref[...]
    ind_r = indr_ref[...]
    ii = jax.lax.broadcasted_iota(jnp.int32, (_B, _B), 0)
    jj = jax.lax.broadcasted_iota(jnp.int32, (_B, _B), 1)
    winner = jnp.min(
        jnp.where((ind_c == ind_r) & (ii > jj), 0.0, 1.0),
        axis=0, keepdims=True)

    gu = gu_ref[...]
    gy = gy_ref[...]
    kk = jax.lax.rem(ind_c, _PK)
    Ug = jnp.zeros((_B, _BIT), jnp.float32)
    Yg = jnp.zeros((_B, _NC), jnp.float32)
    for k in range(_PK):
        mk = kk == k
        Ug = Ug + jnp.where(mk, gu[:, _BIT * k:_BIT * (k + 1)], 0.0)
        Yg = Yg + jnp.where(mk, gy[:, _NC * k:_NC * (k + 1)], 0.0)

    ip_o = jax.lax.dot_general(uh16, Ug.astype(jnp.bfloat16), _DN,
                               preferred_element_type=jnp.float32)
    sd_o = jax.lax.dot_general(y16, Yg.astype(jnp.bfloat16), _DN,
                               preferred_element_type=jnp.float32)
    cs_go, cs_sipo = _colsums(ip_o, sd_o)
    cs_ipo = jax.lax.dot_general(ush, Ug, _DN,
                                 preferred_element_type=jnp.float32)
    c_old = jnp.sum((cs_go + 0.5 * cs_ipo - cs_sipo) * winner)

    ip_n = jax.lax.dot_general(uh16, u.astype(jnp.bfloat16), _DN,
                               preferred_element_type=jnp.float32)
    sd_n = jax.lax.dot_general(y16, y16, _DN,
                               preferred_element_type=jnp.float32)
    cs_gn, cs_sipn = _colsums(ip_n, sd_n)
    cs_ipn = jax.lax.dot_general(ush, u, _DN,
                                 preferred_element_type=jnp.float32)
    c_new = jnp.sum((cs_gn + 0.5 * cs_ipn - cs_sipn) * winner)

    quant = jnp.sum((u - jnp.sign(u)) ** 2) * (_ETA * _NT / _BIT)
    out_ref[...] = (acc_ref[...] + (c_new - c_old + quant)) * (
        1.0 / (_B * _NT))


def _blockdiag(x, width):
    cols = [jnp.pad(x, ((0, 0), (width * k, width * (_PK - 1 - k))))
            for k in range(_PK)]
    return jnp.concatenate(cols, axis=0).astype(jnp.bfloat16)


def kernel(u, y, ind, U, Y):
    ind = ind.astype(jnp.int32)
    ind_c = ind.reshape(_B, 1)
    ind_r = ind.reshape(1, _B)
    ind4 = ind // _PK
    U4 = U.reshape(_NJ, _CBF, _TW)
    Y4 = Y.reshape(_NJ, _CBF, _YW)
    uh4 = _blockdiag(u * 0.5, _BIT)
    y4 = _blockdiag(y, _NC)
    ush = jnp.sum(u * 0.5, axis=0, keepdims=True)
    ush4 = jnp.tile(ush, (1, _PK))

    acc, tabY = pl.pallas_call(
        _main_kernel,
        grid=(_NJ,),
        in_specs=[
            pl.BlockSpec((_B4, _TW), lambda j: (0, 0)),
            pl.BlockSpec((_B4, _YW), lambda j: (0, 0)),
            pl.BlockSpec((1, _TW), lambda j: (0, 0)),
            pl.BlockSpec((1, _CBF, _TW), lambda j: (j, 0, 0)),
            pl.BlockSpec((1, _CBF, _YW), lambda j: (j, 0, 0)),
        ],
        out_specs=(
            pl.BlockSpec((1, 1), lambda j: (0, 0)),
            pl.BlockSpec((1, _CBF, _TW), lambda j: (j, 0, 0)),
        ),
        out_shape=(
            jax.ShapeDtypeStruct((1, 1), jnp.float32),
            jax.ShapeDtypeStruct((_NJ, _CBF, _TW), jnp.float32),
        ),
    )(uh4, y4, ush4, U4, Y4)

    gu, gy = _sc_gather(U4.reshape(_NTF, _TW), tabY.reshape(_NTF, _TW), ind4)

    total = pl.pallas_call(
        _corr_kernel,
        out_shape=jax.ShapeDtypeStruct((1, 1), jnp.float32),
    )(u, y, ind_c, ind_r, gu, gy, acc)
    return total[0, 0]


# speedup vs baseline: 1.3072x; 1.3072x over previous
import functools

import jax
import jax.numpy as jnp
from jax import lax
from jax.experimental import pallas as pl
from jax.experimental.pallas import tpu as pltpu
from jax.experimental.pallas import tpu_sc as plsc

_NT = 50000
_B = 1024
_BIT = 32
_NC = 10
_TW = 128
_ETA = 0.001
_CB = 2000
_NJ = _NT // _CB

_LOG2E = 1.4426950408889634
_LN2 = 0.6931471805599453
_DN = (((1,), (1,)), ((), ()))

_NW = 32
_BPW = _B // _NW


def _sc_gather(table, ind):
    mesh = plsc.VectorSubcoreMesh(core_axis_name="c", subcore_axis_name="s")

    @functools.partial(
        pl.kernel,
        mesh=mesh,
        out_type=jax.ShapeDtypeStruct((_B, _TW), jnp.float32),
        scratch_types=[
            pltpu.VMEM((_BPW,), jnp.int32),
            pltpu.VMEM((_BPW, _TW), jnp.float32),
            pltpu.SemaphoreType.DMA,
        ],
    )
    def gather_k(t_hbm, idx_hbm, g_hbm, idx_v, rows_v, sem):
        wid = lax.axis_index("s") * 2 + lax.axis_index("c")
        base = wid * _BPW
        pltpu.sync_copy(idx_hbm.at[pl.ds(base, _BPW)], idx_v)
        pltpu.async_copy(t_hbm.at[idx_v], rows_v, sem).wait()
        pltpu.sync_copy(rows_v, g_hbm.at[pl.ds(base, _BPW)])

    return gather_k(table, ind)


def _colsums(ip, sd):
    a = jnp.abs(ip)
    lg = jnp.log2(1.0 + jnp.exp2(a * (-_LOG2E)))
    cs_g = (jnp.sum(lg, axis=0, keepdims=True) * _LN2
            + 0.5 * jnp.sum(a, axis=0, keepdims=True))
    cs_sip = jnp.sum(jnp.where(sd > 0, ip, 0.0), axis=0, keepdims=True)
    return cs_g, cs_sip


def _main_kernel(u_ref, y_ref, U_ref, Y_ref, out_ref, tab_ref):
    j = pl.program_id(0)
    uh = u_ref[...] * 0.5
    uh16 = uh.astype(jnp.bfloat16)
    y16 = y_ref[...].astype(jnp.bfloat16)
    ush16 = jnp.sum(uh, axis=0, keepdims=True).astype(jnp.bfloat16)
    Ub16 = U_ref[...]
    Yb16 = Y_ref[...]
    tab_ref[:, 0:_BIT] = Ub16.astype(jnp.float32)
    tab_ref[:, _BIT:_BIT + _NC] = Yb16.astype(jnp.float32)
    ip = jax.lax.dot_general(uh16, Ub16, _DN,
                             preferred_element_type=jnp.float32)
    sd = jax.lax.dot_general(y16, Yb16, _DN,
                             preferred_element_type=jnp.float32)
    cs_g, cs_sip = _colsums(ip, sd)
    cs_ip = jax.lax.dot_general(ush16, Ub16, _DN,
                                preferred_element_type=jnp.float32)
    contrib = jnp.sum(cs_g + 0.5 * cs_ip - cs_sip)

    @pl.when(j == 0)
    def _first():
        out_ref[...] = jnp.full((1, 1), contrib, jnp.float32)

    @pl.when(j != 0)
    def _rest():
        out_ref[...] = out_ref[...] + contrib


def _corr_kernel(u_ref, y_ref, indc_ref, indr_ref, g_ref,
                 acc_ref, out_ref):
    u = u_ref[...]
    uh = u * 0.5
    uh16 = uh.astype(jnp.bfloat16)
    y16 = y_ref[...].astype(jnp.bfloat16)
    ush = jnp.sum(uh, axis=0, keepdims=True)
    ind_c = indc_ref[...]
    ind_r = indr_ref[...]
    ii = jax.lax.broadcasted_iota(jnp.int32, (_B, _B), 0)
    jj = jax.lax.broadcasted_iota(jnp.int32, (_B, _B), 1)
    winner = jnp.min(
        jnp.where((ind_c == ind_r) & (ii > jj), 0.0, 1.0),
        axis=0, keepdims=True)

    g = g_ref[...]
    Ug = g[:, :_BIT]
    ip_o = jax.lax.dot_general(uh16, Ug.astype(jnp.bfloat16), _DN,
                               preferred_element_type=jnp.float32)
    sd_o = jax.lax.dot_general(y16, g[:, _BIT:_BIT + _NC].astype(jnp.bfloat16),
                               _DN, preferred_element_type=jnp.float32)
    cs_go, cs_sipo = _colsums(ip_o, sd_o)
    cs_ipo = jax.lax.dot_general(ush, Ug, _DN,
                                 preferred_element_type=jnp.float32)
    c_old = jnp.sum((cs_go + 0.5 * cs_ipo - cs_sipo) * winner)

    ip_n = jax.lax.dot_general(uh16, u.astype(jnp.bfloat16), _DN,
                               preferred_element_type=jnp.float32)
    sd_n = jax.lax.dot_general(y16, y16, _DN,
                               preferred_element_type=jnp.float32)
    cs_gn, cs_sipn = _colsums(ip_n, sd_n)
    cs_ipn = jax.lax.dot_general(ush, u, _DN,
                                 preferred_element_type=jnp.float32)
    c_new = jnp.sum((cs_gn + 0.5 * cs_ipn - cs_sipn) * winner)

    quant = jnp.sum((u - jnp.sign(u)) ** 2) * (_ETA * _NT / _BIT)
    out_ref[...] = (acc_ref[...] + (c_new - c_old + quant)) * (
        1.0 / (_B * _NT))


def kernel(u, y, ind, U, Y):
    ind = ind.astype(jnp.int32)
    ind_c = ind.reshape(_B, 1)
    ind_r = ind.reshape(1, _B)

    acc, table = pl.pallas_call(
        _main_kernel,
        grid=(_NJ,),
        in_specs=[
            pl.BlockSpec((_B, _BIT), lambda j: (0, 0)),
            pl.BlockSpec((_B, _NC), lambda j: (0, 0)),
            pl.BlockSpec((_CB, _BIT), lambda j: (j, 0)),
            pl.BlockSpec((_CB, _NC), lambda j: (j, 0)),
        ],
        out_specs=(
            pl.BlockSpec((1, 1), lambda j: (0, 0)),
            pl.BlockSpec((_CB, _TW), lambda j: (j, 0)),
        ),
        out_shape=(
            jax.ShapeDtypeStruct((1, 1), jnp.float32),
            jax.ShapeDtypeStruct((_NT, _TW), jnp.float32),
        ),
    )(u, y, U.astype(jnp.bfloat16), Y.astype(jnp.bfloat16))
    g = _sc_gather(table, ind)

    total = pl.pallas_call(
        _corr_kernel,
        out_shape=jax.ShapeDtypeStruct((1, 1), jnp.float32),
    )(u, y, ind_c, ind_r, g, acc)
    return total[0, 0]


# speedup vs baseline: 1.3348x; 1.0211x over previous
import functools

import jax
import jax.numpy as jnp
from jax import lax
from jax.experimental import pallas as pl
from jax.experimental.pallas import tpu as pltpu
from jax.experimental.pallas import tpu_sc as plsc

_NT = 50000
_B = 1024
_BIT = 32
_NC = 10
_TW = 128
_ETA = 0.001
_CB = 5000
_NJ = _NT // _CB

_LOG2E = 1.4426950408889634
_LN2 = 0.6931471805599453
_DN = (((1,), (1,)), ((), ()))

_NW = 32
_BPW = _B // _NW


def _sc_gather(table, ind):
    mesh = plsc.VectorSubcoreMesh(core_axis_name="c", subcore_axis_name="s")

    @functools.partial(
        pl.kernel,
        mesh=mesh,
        out_type=jax.ShapeDtypeStruct((_B, _TW), jnp.float32),
        scratch_types=[
            pltpu.VMEM((_BPW,), jnp.int32),
            pltpu.VMEM((_BPW, _TW), jnp.float32),
            pltpu.SemaphoreType.DMA,
        ],
    )
    def gather_k(t_hbm, idx_hbm, g_hbm, idx_v, rows_v, sem):
        wid = lax.axis_index("s") * 2 + lax.axis_index("c")
        base = wid * _BPW
        pltpu.sync_copy(idx_hbm.at[pl.ds(base, _BPW)], idx_v)
        pltpu.async_copy(t_hbm.at[idx_v], rows_v, sem).wait()
        pltpu.sync_copy(rows_v, g_hbm.at[pl.ds(base, _BPW)])

    return gather_k(table, ind)


def _colsums(ip, sd):
    a = jnp.abs(ip)
    lg = jnp.log2(1.0 + jnp.exp2(a * (-_LOG2E)))
    cs_g = (jnp.sum(lg, axis=0, keepdims=True) * _LN2
            + 0.5 * jnp.sum(a, axis=0, keepdims=True))
    cs_sip = jnp.sum(jnp.where(sd > 0, ip, 0.0), axis=0, keepdims=True)
    return cs_g, cs_sip


def _main_kernel(u_ref, y_ref, U_ref, Y_ref, out_ref, tab_ref):
    j = pl.program_id(0)
    uh = u_ref[...] * 0.5
    uh16 = uh.astype(jnp.bfloat16)
    y16 = y_ref[...].astype(jnp.bfloat16)
    ush16 = jnp.sum(uh, axis=0, keepdims=True).astype(jnp.bfloat16)
    Ub16 = U_ref[...]
    Yb16 = Y_ref[...]
    tab_ref[:, 0:_BIT] = Ub16.astype(jnp.float32)
    tab_ref[:, _BIT:_BIT + _NC] = Yb16.astype(jnp.float32)
    ip = jax.lax.dot_general(uh16, Ub16, _DN,
                             preferred_element_type=jnp.float32)
    sd = jax.lax.dot_general(y16, Yb16, _DN,
                             preferred_element_type=jnp.float32)
    cs_g, cs_sip = _colsums(ip, sd)
    cs_ip = jax.lax.dot_general(ush16, Ub16, _DN,
                                preferred_element_type=jnp.float32)
    contrib = jnp.sum(cs_g + 0.5 * cs_ip - cs_sip)

    @pl.when(j == 0)
    def _first():
        out_ref[...] = jnp.full((1, 1), contrib, jnp.float32)

    @pl.when(j != 0)
    def _rest():
        out_ref[...] = out_ref[...] + contrib


def _corr_kernel(u_ref, y_ref, indc_ref, indr_ref, g_ref,
                 acc_ref, out_ref):
    u = u_ref[...]
    uh = u * 0.5
    uh16 = uh.astype(jnp.bfloat16)
    y16 = y_ref[...].astype(jnp.bfloat16)
    ush = jnp.sum(uh, axis=0, keepdims=True)
    ind_c = indc_ref[...]
    ind_r = indr_ref[...]
    ii = jax.lax.broadcasted_iota(jnp.int32, (_B, _B), 0)
    jj = jax.lax.broadcasted_iota(jnp.int32, (_B, _B), 1)
    winner = jnp.min(
        jnp.where((ind_c == ind_r) & (ii > jj), 0.0, 1.0),
        axis=0, keepdims=True)

    g = g_ref[...]
    Ug = g[:, :_BIT]
    ip_o = jax.lax.dot_general(uh16, Ug.astype(jnp.bfloat16), _DN,
                               preferred_element_type=jnp.float32)
    sd_o = jax.lax.dot_general(y16, g[:, _BIT:_BIT + _NC].astype(jnp.bfloat16),
                               _DN, preferred_element_type=jnp.float32)
    cs_go, cs_sipo = _colsums(ip_o, sd_o)
    cs_ipo = jax.lax.dot_general(ush, Ug, _DN,
                                 preferred_element_type=jnp.float32)
    c_old = jnp.sum((cs_go + 0.5 * cs_ipo - cs_sipo) * winner)

    ip_n = jax.lax.dot_general(uh16, u.astype(jnp.bfloat16), _DN,
                               preferred_element_type=jnp.float32)
    sd_n = jax.lax.dot_general(y16, y16, _DN,
                               preferred_element_type=jnp.float32)
    cs_gn, cs_sipn = _colsums(ip_n, sd_n)
    cs_ipn = jax.lax.dot_general(ush, u, _DN,
                                 preferred_element_type=jnp.float32)
    c_new = jnp.sum((cs_gn + 0.5 * cs_ipn - cs_sipn) * winner)

    quant = jnp.sum((u - jnp.sign(u)) ** 2) * (_ETA * _NT / _BIT)
    out_ref[...] = (acc_ref[...] + (c_new - c_old + quant)) * (
        1.0 / (_B * _NT))


def kernel(u, y, ind, U, Y):
    ind = ind.astype(jnp.int32)
    ind_c = ind.reshape(_B, 1)
    ind_r = ind.reshape(1, _B)

    acc, table = pl.pallas_call(
        _main_kernel,
        grid=(_NJ,),
        in_specs=[
            pl.BlockSpec((_B, _BIT), lambda j: (0, 0)),
            pl.BlockSpec((_B, _NC), lambda j: (0, 0)),
            pl.BlockSpec((_CB, _BIT), lambda j: (j, 0)),
            pl.BlockSpec((_CB, _NC), lambda j: (j, 0)),
        ],
        out_specs=(
            pl.BlockSpec((1, 1), lambda j: (0, 0)),
            pl.BlockSpec((_CB, _TW), lambda j: (j, 0)),
        ),
        out_shape=(
            jax.ShapeDtypeStruct((1, 1), jnp.float32),
            jax.ShapeDtypeStruct((_NT, _TW), jnp.float32),
        ),
    )(u, y, U.astype(jnp.bfloat16), Y.astype(jnp.bfloat16))
    g = _sc_gather(table, ind)

    total = pl.pallas_call(
        _corr_kernel,
        out_shape=jax.ShapeDtypeStruct((1, 1), jnp.float32),
    )(u, y, ind_c, ind_r, g, acc)
    return total[0, 0]
